# baseline (speedup 1.0000x reference)
import jax
import jax.numpy as jnp
from jax import lax
from jax.experimental import pallas as pl
from jax.experimental.pallas import tpu as pltpu
from jax.experimental.pallas import tpu_sc as plsc

_D = 64
_L = 16
_NC = 2
_NS = 16
_NW = _NC * _NS
_NBUF = 4
_SCALE = 8.0


def _make_sc_gather(n_rows: int, seq: int):
  mesh = plsc.VectorSubcoreMesh(core_axis_name="c", subcore_axis_name="s")
  r_per_w = n_rows // _NW

  def body(idx_hbm, table_hbm, out_hbm, idx_v, buf, *sems):
    gsems = sems[:_NBUF]
    osems = sems[_NBUF:]
    wid = lax.axis_index("s") * _NC + lax.axis_index("c")
    base = wid * r_per_w

    pltpu.sync_copy(idx_hbm.at[pl.ds(base, r_per_w)], idx_v)

    def g_copy(j, b):
      return pltpu.make_async_copy(
          table_hbm.at[idx_v.at[j]], buf.at[b], gsems[b])

    def o_copy(j, b):
      return pltpu.make_async_copy(
          buf.at[b], out_hbm.at[base + j], osems[b])

    for b in range(_NBUF):
      g_copy(b, b).start()

    def outer(io, carry):
      jo = io * _NBUF
      for b in range(_NBUF):
        j = jo + b
        g_copy(j, b).wait()
        o_copy(j, b).start()

        @pl.when(j + _NBUF < r_per_w)
        def _():
          o_copy(j, b).wait()
          g_copy(j + _NBUF, b).start()
      return carry

    lax.fori_loop(0, r_per_w // _NBUF, outer, 0)

    for b in range(_NBUF):
      o_copy(r_per_w - _NBUF + b, b).wait()

  return pl.kernel(
      body,
      mesh=mesh,
      out_type=jax.ShapeDtypeStruct((n_rows, seq, _D), jnp.float32),
      scratch_types=[
          pltpu.VMEM((r_per_w, seq), jnp.int32),
          pltpu.VMEM((_NBUF, seq, _D), jnp.float32),
      ] + [pltpu.SemaphoreType.DMA] * (2 * _NBUF),
      compiler_params=pltpu.CompilerParams(use_tc_tiling_on_sc=False),
  )


def kernel(x, table):
  n_rows, seq = x.shape
  assert n_rows % (_NW * _NBUF) == 0 and seq % 4 == 0
  raw = _make_sc_gather(n_rows, seq)(x.astype(jnp.int32), table)
  return raw * jnp.float32(_SCALE)


# speedup vs baseline: 1.2117x; 1.2117x over previous
import jax
import jax.numpy as jnp
from jax import lax
from jax.experimental import pallas as pl
from jax.experimental.pallas import tpu as pltpu
from jax.experimental.pallas import tpu_sc as plsc

_D = 64
_L = 16
_NC = 2
_NS = 16
_NW = _NC * _NS
_NBUF = 4
_SCALE = 8.0


def _make_sc_gather(n_rows: int, seq: int):
  mesh = plsc.VectorSubcoreMesh(core_axis_name="c", subcore_axis_name="s")
  r_per_w = n_rows // _NW

  def body(idx_hbm, table_hbm, out_hbm, idx_v, buf, *sems):
    gsems = sems[:_NBUF]
    osems = sems[_NBUF:]
    wid = lax.axis_index("s") * _NC + lax.axis_index("c")
    base = wid * r_per_w

    pltpu.sync_copy(idx_hbm.at[pl.ds(base, r_per_w)], idx_v)

    def g_copy(j, b):
      return pltpu.make_async_copy(
          table_hbm.at[idx_v.at[j]], buf.at[b], gsems[b])

    def o_copy(j, b):
      return pltpu.make_async_copy(
          buf.at[b], out_hbm.at[base + j], osems[b])

    for b in range(_NBUF):
      g_copy(b, b).start()

    def outer(io, carry):
      jo = io * _NBUF
      for b in range(_NBUF):
        j = jo + b
        g_copy(j, b).wait()

        def srow(i4, c):
          for u in range(4):
            i = i4 * 4 + u
            for l in range(_D // _L):
              s = pl.ds(l * _L, _L)
              buf[b, i, s] = buf[b, i, s] * _SCALE
          return c
        lax.fori_loop(0, seq // 4, srow, 0)

        o_copy(j, b).start()

        @pl.when(j + _NBUF < r_per_w)
        def _():
          o_copy(j, b).wait()
          g_copy(j + _NBUF, b).start()
      return carry

    lax.fori_loop(0, r_per_w // _NBUF, outer, 0)

    for b in range(_NBUF):
      o_copy(r_per_w - _NBUF + b, b).wait()

  return pl.kernel(
      body,
      mesh=mesh,
      out_type=jax.ShapeDtypeStruct((n_rows, seq, _D), jnp.float32),
      scratch_types=[
          pltpu.VMEM((r_per_w, seq), jnp.int32),
          pltpu.VMEM((_NBUF, seq, _D), jnp.float32),
      ] + [pltpu.SemaphoreType.DMA] * (2 * _NBUF),
      compiler_params=pltpu.CompilerParams(use_tc_tiling_on_sc=False),
  )


def kernel(x, table):
  n_rows, seq = x.shape
  assert n_rows % (_NW * _NBUF) == 0 and seq % 4 == 0
  return _make_sc_gather(n_rows, seq)(x.astype(jnp.int32), table)
